# Initial kernel scaffold; baseline (speedup 1.0000x reference)
import jax
import jax.numpy as jnp
from jax.experimental import pallas as pl


def kernel(z, embed):
    raise NotImplementedError("write your pallas kernel here")


# speedup vs baseline: 1.1275x; 1.1275x over previous
import jax
import jax.numpy as jnp
from jax.experimental import pallas as pl


def _rvq_body(z_ref, embed_ref, codes_ref, zq_ref, res_ref, loss_ref, usage_ref):
    i = pl.program_id(0)
    nblocks = pl.num_programs(0)
    Bb, D = z_ref.shape
    R, K, _ = embed_ref.shape

    @pl.when(i == 0)
    def _init():
        loss_ref[...] = jnp.zeros_like(loss_ref)
        usage_ref[...] = jnp.zeros_like(usage_ref)

    r = z_ref[...]
    z = r
    zq = jnp.zeros_like(r)
    iota_k = jax.lax.broadcasted_iota(jnp.int32, (Bb, K), 1)

    counts = []
    for s in range(R):
        cb = embed_ref[s]
        xe = jax.lax.dot_general(
            r.astype(jnp.bfloat16), cb.astype(jnp.bfloat16),
            (((1,), (1,)), ((), ())),
            preferred_element_type=jnp.float32)
        x2 = jnp.sum(r * r, axis=1, keepdims=True)
        e2 = jnp.sum(cb * cb, axis=1)[None, :]
        dist = x2 + e2 - 2.0 * xe
        m = jnp.min(dist, axis=1, keepdims=True)
        codes_s = jnp.min(jnp.where(dist == m, iota_k, K), axis=1)
        onehot = (iota_k == codes_s[:, None]).astype(jnp.float32)
        quant = jnp.dot(onehot, cb, preferred_element_type=jnp.float32,
                        precision=jax.lax.Precision.HIGHEST)
        codes_ref[s, :] = codes_s
        counts.append(jnp.sum(onehot, axis=0))
        zq = zq + quant
        r = r - quant

    zq_ref[...] = zq
    res_ref[...] = r
    usage_ref[...] = usage_ref[...] + jnp.stack(counts, axis=0)
    diff = zq - z
    loss_ref[...] = loss_ref[...] + jnp.sum(diff * diff)[None, None]

    @pl.when(i == nblocks - 1)
    def _finalize():
        total = Bb * nblocks
        loss_ref[...] = loss_ref[...] / (total * D)
        usage_ref[...] = usage_ref[...] / total


def kernel(z, embed):
    B, D = z.shape
    R, K, _ = embed.shape
    Bb = 512
    nblocks = B // Bb

    codes_t, zq, res, loss, usage = pl.pallas_call(
        _rvq_body,
        grid=(nblocks,),
        in_specs=[
            pl.BlockSpec((Bb, D), lambda i: (i, 0)),
            pl.BlockSpec((R, K, D), lambda i: (0, 0, 0)),
        ],
        out_specs=[
            pl.BlockSpec((R, Bb), lambda i: (0, i)),
            pl.BlockSpec((Bb, D), lambda i: (i, 0)),
            pl.BlockSpec((Bb, D), lambda i: (i, 0)),
            pl.BlockSpec((1, 1), lambda i: (0, 0)),
            pl.BlockSpec((R, K), lambda i: (0, 0)),
        ],
        out_shape=[
            jax.ShapeDtypeStruct((R, B), jnp.int32),
            jax.ShapeDtypeStruct((B, D), jnp.float32),
            jax.ShapeDtypeStruct((B, D), jnp.float32),
            jax.ShapeDtypeStruct((1, 1), jnp.float32),
            jax.ShapeDtypeStruct((R, K), jnp.float32),
        ],
    )(z, embed)

    codes = codes_t.T
    recon_loss = loss[0, 0]
    return codes, zq, res, recon_loss, usage


# speedup vs baseline: 2.0681x; 1.8342x over previous
import jax
import jax.numpy as jnp
from jax.experimental import pallas as pl


def _rvq_body(z_ref, embed_ref, ehi_ref, emid_ref, elo_ref,
              codes_ref, zq_ref, res_ref, loss_ref, usage_ref):
    i = pl.program_id(0)
    nblocks = pl.num_programs(0)
    Bb, D = z_ref.shape
    R, K, _ = ehi_ref.shape

    @pl.when(i == 0)
    def _init():
        loss_ref[...] = jnp.zeros_like(loss_ref)
        usage_ref[...] = jnp.zeros_like(usage_ref)

    r = z_ref[...]
    z = r
    zq = jnp.zeros_like(r)
    iota_f = jax.lax.broadcasted_iota(jnp.int32, (Bb, K), 1).astype(jnp.float32)

    def dot_nt(a, b):
        return jax.lax.dot_general(a, b, (((1,), (1,)), ((), ())),
                                   preferred_element_type=jnp.float32)

    counts = []
    for s in range(R):
        cb = embed_ref[s]
        cb_hi = ehi_ref[s]
        cb_mid = emid_ref[s]
        cb_lo = elo_ref[s]
        xe = dot_nt(r.astype(jnp.bfloat16), cb_hi)
        x2 = jnp.sum(r * r, axis=1, keepdims=True)
        e2 = jnp.sum(cb * cb, axis=1)[None, :]
        dist = x2 + e2 - 2.0 * xe
        m = jnp.min(dist, axis=1, keepdims=True)
        codes_f = jnp.min(jnp.where(dist == m, iota_f, float(K)), axis=1)
        codes_s = codes_f.astype(jnp.int32)
        onehot = (iota_f == codes_f[:, None]).astype(jnp.bfloat16)
        quant = (jnp.dot(onehot, cb_hi, preferred_element_type=jnp.float32)
                 + jnp.dot(onehot, cb_mid, preferred_element_type=jnp.float32)
                 ) + jnp.dot(onehot, cb_lo, preferred_element_type=jnp.float32)
        codes_ref[s, :] = codes_s
        counts.append(jnp.sum(onehot, axis=0, dtype=jnp.float32))
        zq = zq + quant
        r = r - quant

    zq_ref[...] = zq
    res_ref[...] = r
    usage_ref[...] = usage_ref[...] + jnp.stack(counts, axis=0)
    diff = zq - z
    loss_ref[...] = loss_ref[...] + jnp.sum(diff * diff)[None, None]

    @pl.when(i == nblocks - 1)
    def _finalize():
        total = Bb * nblocks
        loss_ref[...] = loss_ref[...] / (total * D)
        usage_ref[...] = usage_ref[...] / total


def kernel(z, embed):
    B, D = z.shape
    R, K, _ = embed.shape
    Bb = 1024
    nblocks = B // Bb

    e_hi = embed.astype(jnp.bfloat16)
    r1 = embed - e_hi.astype(jnp.float32)
    e_mid = r1.astype(jnp.bfloat16)
    e_lo = (r1 - e_mid.astype(jnp.float32)).astype(jnp.bfloat16)

    codes_t, zq, res, loss, usage = pl.pallas_call(
        _rvq_body,
        grid=(nblocks,),
        in_specs=[
            pl.BlockSpec((Bb, D), lambda i: (i, 0)),
            pl.BlockSpec((R, K, D), lambda i: (0, 0, 0)),
            pl.BlockSpec((R, K, D), lambda i: (0, 0, 0)),
            pl.BlockSpec((R, K, D), lambda i: (0, 0, 0)),
            pl.BlockSpec((R, K, D), lambda i: (0, 0, 0)),
        ],
        out_specs=[
            pl.BlockSpec((R, Bb), lambda i: (0, i)),
            pl.BlockSpec((Bb, D), lambda i: (i, 0)),
            pl.BlockSpec((Bb, D), lambda i: (i, 0)),
            pl.BlockSpec((1, 1), lambda i: (0, 0)),
            pl.BlockSpec((R, K), lambda i: (0, 0)),
        ],
        out_shape=[
            jax.ShapeDtypeStruct((R, B), jnp.int32),
            jax.ShapeDtypeStruct((B, D), jnp.float32),
            jax.ShapeDtypeStruct((B, D), jnp.float32),
            jax.ShapeDtypeStruct((1, 1), jnp.float32),
            jax.ShapeDtypeStruct((R, K), jnp.float32),
        ],
    )(z, embed, e_hi, e_mid, e_lo)

    codes = codes_t.T
    recon_loss = loss[0, 0]
    return codes, zq, res, recon_loss, usage


# speedup vs baseline: 2.1828x; 1.0554x over previous
import jax
import jax.numpy as jnp
from jax.experimental import pallas as pl


def _rvq_body(z_ref, e2_ref, ehi_ref, emid_ref, elo_ref,
              codes_ref, zq_ref, res_ref, loss_ref, usage_ref):
    i = pl.program_id(0)
    nblocks = pl.num_programs(0)
    Bb, D = z_ref.shape
    R, K, _ = ehi_ref.shape

    @pl.when(i == 0)
    def _init():
        loss_ref[...] = jnp.zeros_like(loss_ref)
        usage_ref[...] = jnp.zeros_like(usage_ref)

    r = z_ref[...]
    z = r
    zq = jnp.zeros_like(r)
    iota_f = jax.lax.broadcasted_iota(jnp.int32, (Bb, K), 1).astype(jnp.float32)
    C = 128
    nc = K // C

    def dot_nt(a, b):
        return jax.lax.dot_general(a, b, (((1,), (1,)), ((), ())),
                                   preferred_element_type=jnp.float32)

    ones_row = jnp.ones((1, Bb), dtype=jnp.bfloat16)
    counts = []
    for s in range(R):
        cb_hi = ehi_ref[s]
        cb_mid = emid_ref[s]
        cb_lo = elo_ref[s]
        xe2 = dot_nt((-2.0 * r).astype(jnp.bfloat16), cb_hi)
        x2 = jnp.sum(r * r, axis=1, keepdims=True)
        e2 = e2_ref[s]
        v = None
        for c in range(nc):
            d_c = (x2 + e2[:, c * C:(c + 1) * C]) + xe2[:, c * C:(c + 1) * C]
            i_c = (jax.lax.broadcasted_iota(jnp.int32, (Bb, C), 1)
                   + c * C).astype(jnp.float32)
            if v is None:
                v, idx = d_c, i_c
            else:
                lt = d_c < v
                v = jnp.where(lt, d_c, v)
                idx = jnp.where(lt, i_c, idx)
        m = jnp.min(v, axis=1, keepdims=True)
        codes_f = jnp.min(jnp.where(v == m, idx, float(K)),
                          axis=1, keepdims=True)
        onehot = (iota_f == codes_f).astype(jnp.bfloat16)
        quant = (jnp.dot(onehot, cb_hi, preferred_element_type=jnp.float32)
                 + jnp.dot(onehot, cb_mid, preferred_element_type=jnp.float32)
                 ) + jnp.dot(onehot, cb_lo, preferred_element_type=jnp.float32)
        codes_ref[:, s:s + 1] = codes_f.astype(jnp.int32)
        counts.append(jnp.dot(ones_row, onehot,
                              preferred_element_type=jnp.float32)[0])
        zq = zq + quant
        r = r - quant

    zq_ref[...] = zq
    res_ref[...] = r
    usage_ref[...] = usage_ref[...] + jnp.stack(counts, axis=0)
    diff = zq - z
    loss_ref[...] = loss_ref[...] + jnp.sum(diff * diff)[None, None]

    @pl.when(i == nblocks - 1)
    def _finalize():
        total = Bb * nblocks
        loss_ref[...] = loss_ref[...] / (total * D)
        usage_ref[...] = usage_ref[...] / total


def kernel(z, embed):
    B, D = z.shape
    R, K, _ = embed.shape
    Bb = 1024
    nblocks = B // Bb

    e_hi = embed.astype(jnp.bfloat16)
    r1 = embed - e_hi.astype(jnp.float32)
    e_mid = r1.astype(jnp.bfloat16)
    e_lo = (r1 - e_mid.astype(jnp.float32)).astype(jnp.bfloat16)
    e2 = jnp.sum(embed * embed, axis=2)[:, None, :]

    codes, zq, res, loss, usage = pl.pallas_call(
        _rvq_body,
        grid=(nblocks,),
        in_specs=[
            pl.BlockSpec((Bb, D), lambda i: (i, 0)),
            pl.BlockSpec((R, 1, K), lambda i: (0, 0, 0)),
            pl.BlockSpec((R, K, D), lambda i: (0, 0, 0)),
            pl.BlockSpec((R, K, D), lambda i: (0, 0, 0)),
            pl.BlockSpec((R, K, D), lambda i: (0, 0, 0)),
        ],
        out_specs=[
            pl.BlockSpec((Bb, R), lambda i: (i, 0)),
            pl.BlockSpec((Bb, D), lambda i: (i, 0)),
            pl.BlockSpec((Bb, D), lambda i: (i, 0)),
            pl.BlockSpec((1, 1), lambda i: (0, 0)),
            pl.BlockSpec((R, K), lambda i: (0, 0)),
        ],
        out_shape=[
            jax.ShapeDtypeStruct((B, R), jnp.int32),
            jax.ShapeDtypeStruct((B, D), jnp.float32),
            jax.ShapeDtypeStruct((B, D), jnp.float32),
            jax.ShapeDtypeStruct((1, 1), jnp.float32),
            jax.ShapeDtypeStruct((R, K), jnp.float32),
        ],
    )(z, e2, e_hi, e_mid, e_lo)

    recon_loss = loss[0, 0]
    return codes, zq, res, recon_loss, usage


# speedup vs baseline: 2.9296x; 1.3422x over previous
import jax
import jax.numpy as jnp
from jax.experimental import pallas as pl


def _rvq_body(z_ref, e2_ref, ehi_ref, emid_ref, elo_ref,
              codes_ref, zq_ref, res_ref, loss_ref, usage_ref):
    i = pl.program_id(0)
    nblocks = pl.num_programs(0)
    Bb, D = z_ref.shape
    R, K, _ = ehi_ref.shape

    @pl.when(i == 0)
    def _init():
        loss_ref[...] = jnp.zeros_like(loss_ref)
        usage_ref[...] = jnp.zeros_like(usage_ref)

    r = z_ref[...]
    z = r
    zq = jnp.zeros_like(r)
    iota_f = jax.lax.broadcasted_iota(jnp.int32, (Bb, K), 1).astype(jnp.float32)
    C = 128
    nc = K // C

    def dot_nt(a, b):
        return jax.lax.dot_general(a, b, (((1,), (1,)), ((), ())),
                                   preferred_element_type=jnp.float32)

    H = Bb // 2
    ones_row = jnp.ones((1, H), dtype=jnp.bfloat16)
    iota_h = iota_f[:H]

    def stage(rh, cb_hi, cb_mid, cb_lo, e2):
        xe2 = dot_nt((-2.0 * rh).astype(jnp.bfloat16), cb_hi)
        x2 = jnp.sum(rh * rh, axis=1, keepdims=True)
        v = None
        for c in range(nc):
            d_c = (x2 + e2[:, c * C:(c + 1) * C]) + xe2[:, c * C:(c + 1) * C]
            i_c = (jax.lax.broadcasted_iota(jnp.int32, (H, C), 1)
                   + c * C).astype(jnp.float32)
            if v is None:
                v, idx = d_c, i_c
            else:
                lt = d_c < v
                v = jnp.where(lt, d_c, v)
                idx = jnp.where(lt, i_c, idx)
        m = jnp.min(v, axis=1, keepdims=True)
        codes_f = jnp.min(jnp.where(v == m, idx, float(K)),
                          axis=1, keepdims=True)
        onehot = (iota_h == codes_f).astype(jnp.bfloat16)
        quant = (jnp.dot(onehot, cb_hi, preferred_element_type=jnp.float32)
                 + jnp.dot(onehot, cb_mid, preferred_element_type=jnp.float32)
                 ) + jnp.dot(onehot, cb_lo, preferred_element_type=jnp.float32)
        cnt = jnp.dot(ones_row, onehot, preferred_element_type=jnp.float32)
        return codes_f, quant, cnt

    counts = []
    r0, r1 = r[:H], r[H:]
    zq0, zq1 = zq[:H], zq[H:]
    for s in range(R):
        cb_hi = ehi_ref[s]
        cb_mid = emid_ref[s]
        cb_lo = elo_ref[s]
        e2 = e2_ref[s]
        codes0, quant0, cnt0 = stage(r0, cb_hi, cb_mid, cb_lo, e2)
        codes1, quant1, cnt1 = stage(r1, cb_hi, cb_mid, cb_lo, e2)
        codes_ref[:H, s:s + 1] = codes0.astype(jnp.int32)
        codes_ref[H:, s:s + 1] = codes1.astype(jnp.int32)
        counts.append((cnt0 + cnt1)[0])
        zq0, zq1 = zq0 + quant0, zq1 + quant1
        r0, r1 = r0 - quant0, r1 - quant1

    zq_ref[:H] = zq0
    zq_ref[H:] = zq1
    res_ref[:H] = r0
    res_ref[H:] = r1
    zq = jnp.concatenate([zq0, zq1], axis=0)
    r = jnp.concatenate([r0, r1], axis=0)
    usage_ref[...] = usage_ref[...] + jnp.stack(counts, axis=0)
    diff = zq - z
    loss_ref[...] = loss_ref[...] + jnp.sum(diff * diff)[None, None]

    @pl.when(i == nblocks - 1)
    def _finalize():
        total = Bb * nblocks
        loss_ref[...] = loss_ref[...] / (total * D)
        usage_ref[...] = usage_ref[...] / total


def kernel(z, embed):
    B, D = z.shape
    R, K, _ = embed.shape
    Bb = 1024
    nblocks = B // Bb

    e_hi = embed.astype(jnp.bfloat16)
    r1 = embed - e_hi.astype(jnp.float32)
    e_mid = r1.astype(jnp.bfloat16)
    e_lo = (r1 - e_mid.astype(jnp.float32)).astype(jnp.bfloat16)
    e2 = jnp.sum(embed * embed, axis=2)[:, None, :]

    codes, zq, res, loss, usage = pl.pallas_call(
        _rvq_body,
        grid=(nblocks,),
        in_specs=[
            pl.BlockSpec((Bb, D), lambda i: (i, 0)),
            pl.BlockSpec((R, 1, K), lambda i: (0, 0, 0)),
            pl.BlockSpec((R, K, D), lambda i: (0, 0, 0)),
            pl.BlockSpec((R, K, D), lambda i: (0, 0, 0)),
            pl.BlockSpec((R, K, D), lambda i: (0, 0, 0)),
        ],
        out_specs=[
            pl.BlockSpec((Bb, R), lambda i: (i, 0)),
            pl.BlockSpec((Bb, D), lambda i: (i, 0)),
            pl.BlockSpec((Bb, D), lambda i: (i, 0)),
            pl.BlockSpec((1, 1), lambda i: (0, 0)),
            pl.BlockSpec((R, K), lambda i: (0, 0)),
        ],
        out_shape=[
            jax.ShapeDtypeStruct((B, R), jnp.int32),
            jax.ShapeDtypeStruct((B, D), jnp.float32),
            jax.ShapeDtypeStruct((B, D), jnp.float32),
            jax.ShapeDtypeStruct((1, 1), jnp.float32),
            jax.ShapeDtypeStruct((R, K), jnp.float32),
        ],
    )(z, e2, e_hi, e_mid, e_lo)

    recon_loss = loss[0, 0]
    return codes, zq, res, recon_loss, usage


# speedup vs baseline: 2.9319x; 1.0008x over previous
import jax
import jax.numpy as jnp
from jax.experimental import pallas as pl
from jax.experimental.pallas import tpu as pltpu


def _rvq_body(z_ref, embed_ref,
              codes_ref, zq_ref, res_ref, loss_ref, usage_ref,
              e2_ref, ehi_ref, emid_ref, elo_ref):
    i = pl.program_id(0)
    nblocks = pl.num_programs(0)
    Bb, D = z_ref.shape
    R, K, _ = embed_ref.shape

    @pl.when(i == 0)
    def _init():
        loss_ref[...] = jnp.zeros_like(loss_ref)
        usage_ref[...] = jnp.zeros_like(usage_ref)
        for s in range(R):
            cb = embed_ref[s]
            hi = cb.astype(jnp.bfloat16)
            t = cb - hi.astype(jnp.float32)
            mid = t.astype(jnp.bfloat16)
            lo = (t - mid.astype(jnp.float32)).astype(jnp.bfloat16)
            ehi_ref[s] = hi
            emid_ref[s] = mid
            elo_ref[s] = lo
            e2_ref[s] = jnp.sum(cb * cb, axis=1)[None, :]

    r = z_ref[...]
    z = r
    zq = jnp.zeros_like(r)
    iota_f = jax.lax.broadcasted_iota(jnp.int32, (Bb, K), 1).astype(jnp.float32)
    C = 128
    nc = K // C

    def dot_nt(a, b):
        return jax.lax.dot_general(a, b, (((1,), (1,)), ((), ())),
                                   preferred_element_type=jnp.float32)

    H = Bb // 2
    ones_row = jnp.ones((1, H), dtype=jnp.bfloat16)
    iota_h = iota_f[:H]

    def stage(rh, cb_hi, cb_mid, cb_lo, e2):
        xe2 = dot_nt((-2.0 * rh).astype(jnp.bfloat16), cb_hi)
        x2 = jnp.sum(rh * rh, axis=1, keepdims=True)
        v = None
        for c in range(nc):
            d_c = (x2 + e2[:, c * C:(c + 1) * C]) + xe2[:, c * C:(c + 1) * C]
            i_c = (jax.lax.broadcasted_iota(jnp.int32, (H, C), 1)
                   + c * C).astype(jnp.float32)
            if v is None:
                v, idx = d_c, i_c
            else:
                lt = d_c < v
                v = jnp.where(lt, d_c, v)
                idx = jnp.where(lt, i_c, idx)
        m = jnp.min(v, axis=1, keepdims=True)
        codes_f = jnp.min(jnp.where(v == m, idx, float(K)),
                          axis=1, keepdims=True)
        onehot = (iota_h == codes_f).astype(jnp.bfloat16)
        quant = (jnp.dot(onehot, cb_hi, preferred_element_type=jnp.float32)
                 + jnp.dot(onehot, cb_mid, preferred_element_type=jnp.float32)
                 ) + jnp.dot(onehot, cb_lo, preferred_element_type=jnp.float32)
        cnt = jnp.dot(ones_row, onehot, preferred_element_type=jnp.float32)
        return codes_f, quant, cnt

    counts = []
    r0, r1 = r[:H], r[H:]
    zq0, zq1 = zq[:H], zq[H:]
    for s in range(R):
        cb_hi = ehi_ref[s]
        cb_mid = emid_ref[s]
        cb_lo = elo_ref[s]
        e2 = e2_ref[s]
        codes0, quant0, cnt0 = stage(r0, cb_hi, cb_mid, cb_lo, e2)
        codes1, quant1, cnt1 = stage(r1, cb_hi, cb_mid, cb_lo, e2)
        codes_ref[:H, s:s + 1] = codes0.astype(jnp.int32)
        codes_ref[H:, s:s + 1] = codes1.astype(jnp.int32)
        counts.append((cnt0 + cnt1)[0])
        zq0, zq1 = zq0 + quant0, zq1 + quant1
        r0, r1 = r0 - quant0, r1 - quant1

    zq_ref[:H] = zq0
    zq_ref[H:] = zq1
    res_ref[:H] = r0
    res_ref[H:] = r1
    zq = jnp.concatenate([zq0, zq1], axis=0)
    r = jnp.concatenate([r0, r1], axis=0)
    usage_ref[...] = usage_ref[...] + jnp.stack(counts, axis=0)
    diff = zq - z
    loss_ref[...] = loss_ref[...] + jnp.sum(diff * diff)[None, None]

    @pl.when(i == nblocks - 1)
    def _finalize():
        total = Bb * nblocks
        loss_ref[...] = loss_ref[...] / (total * D)
        usage_ref[...] = usage_ref[...] / total


def kernel(z, embed):
    B, D = z.shape
    R, K, _ = embed.shape
    Bb = 1024
    nblocks = B // Bb

    codes, zq, res, loss, usage = pl.pallas_call(
        _rvq_body,
        grid=(nblocks,),
        in_specs=[
            pl.BlockSpec((Bb, D), lambda i: (i, 0)),
            pl.BlockSpec((R, K, D), lambda i: (0, 0, 0)),
        ],
        out_specs=[
            pl.BlockSpec((Bb, R), lambda i: (i, 0)),
            pl.BlockSpec((Bb, D), lambda i: (i, 0)),
            pl.BlockSpec((Bb, D), lambda i: (i, 0)),
            pl.BlockSpec((1, 1), lambda i: (0, 0)),
            pl.BlockSpec((R, K), lambda i: (0, 0)),
        ],
        out_shape=[
            jax.ShapeDtypeStruct((B, R), jnp.int32),
            jax.ShapeDtypeStruct((B, D), jnp.float32),
            jax.ShapeDtypeStruct((B, D), jnp.float32),
            jax.ShapeDtypeStruct((1, 1), jnp.float32),
            jax.ShapeDtypeStruct((R, K), jnp.float32),
        ],
        scratch_shapes=[
            pltpu.VMEM((R, 1, K), jnp.float32),
            pltpu.VMEM((R, K, D), jnp.bfloat16),
            pltpu.VMEM((R, K, D), jnp.bfloat16),
            pltpu.VMEM((R, K, D), jnp.bfloat16),
        ],
    )(z, embed)

    recon_loss = loss[0, 0]
    return codes, zq, res, recon_loss, usage
